# Initial kernel scaffold; baseline (speedup 1.0000x reference)
import jax
import jax.numpy as jnp
from jax.experimental import pallas as pl


def kernel(x_user, x_item, edge_index_ui, edge_index_iu, W0_ui_s, W0_ui_t, W0_iu_s, W0_iu_t, W1_ui_s, W1_ui_t, W1_iu_s, W1_iu_t, Wl_u, bl_u, Wl_i, bl_i):
    raise NotImplementedError("write your pallas kernel here")


# speedup vs baseline: 10.2117x; 10.2117x over previous
import functools

import jax
import jax.numpy as jnp
from jax import lax
from jax.experimental import pallas as pl
from jax.experimental.pallas import tpu as pltpu
from jax.experimental.pallas import tpu_sc as plsc

_N = 10000
_D = 128
_E = 320000
_C = 128
_NCH = _E // _C
_NSUB = 16
_KMAX = (_NCH + _NSUB - 1) // _NSUB
_RPT = _N // _NSUB
_ZC = 624

_mesh = plsc.VectorSubcoreMesh(core_axis_name="c", subcore_axis_name="s")


@functools.partial(
    pl.kernel,
    out_type=tuple(jax.ShapeDtypeStruct((_N,), jnp.float32) for _ in range(4)),
    mesh=_mesh,
    scratch_types=[
        pltpu.VMEM((_C,), jnp.int32),
        pltpu.VMEM((_C,), jnp.float32),
        pltpu.VMEM((_ZC,), jnp.float32),
        pltpu.VMEM_SHARED((_N,), jnp.float32),
        pltpu.VMEM_SHARED((_N,), jnp.float32),
    ],
)
def _deg_kernel(s0, t0, s1, t1, ds0, dt0, ds1, dt1, idx, ones, zbuf, sh_s, sh_t):
    cid = lax.axis_index("c")
    sid = lax.axis_index("s")

    for i in range(_C // 16):
        ones[pl.ds(i * 16, 16)] = jnp.full((16,), 1.0, jnp.float32)
    for i in range(_ZC // 16):
        zbuf[pl.ds(i * 16, 16)] = jnp.zeros((16,), jnp.float32)

    base = sid * _ZC
    pltpu.sync_copy(zbuf, sh_s.at[pl.ds(base, _ZC)])
    pltpu.sync_copy(zbuf, sh_t.at[pl.ds(base, _ZC)])

    @pl.when(sid == _NSUB - 1)
    def _():
        tail = _NSUB * _ZC
        pltpu.sync_copy(zbuf.at[pl.ds(0, _N - tail)], sh_s.at[pl.ds(tail, _N - tail)])
        pltpu.sync_copy(zbuf.at[pl.ds(0, _N - tail)], sh_t.at[pl.ds(tail, _N - tail)])

    plsc.subcore_barrier()

    for c, (s_h, t_h) in enumerate(((s0, t0), (s1, t1))):
        @pl.when(cid == c)
        def _():
            def body(k, carry):
                g = k * _NSUB + sid

                @pl.when(g < _NCH)
                def _():
                    pltpu.sync_copy(s_h.at[pl.ds(g * _C, _C)], idx)
                    pltpu.sync_copy(ones, sh_s.at[idx], add=True)
                    pltpu.sync_copy(t_h.at[pl.ds(g * _C, _C)], idx)
                    pltpu.sync_copy(ones, sh_t.at[idx], add=True)
                return carry

            lax.fori_loop(0, _KMAX, body, 0)

    plsc.subcore_barrier()

    for c, (o_s, o_t) in enumerate(((ds0, dt0), (ds1, dt1))):
        @pl.when(cid == c)
        def _():
            for sh, out in ((sh_s, o_s), (sh_t, o_t)):
                pltpu.sync_copy(sh.at[pl.ds(base, _ZC)], zbuf)
                pltpu.sync_copy(zbuf, out.at[pl.ds(base, _ZC)])

                @pl.when(sid == _NSUB - 1)
                def _():
                    tail = _NSUB * _ZC
                    pltpu.sync_copy(sh.at[pl.ds(tail, _N - tail)], zbuf.at[pl.ds(0, _N - tail)])
                    pltpu.sync_copy(zbuf.at[pl.ds(0, _N - tail)], out.at[pl.ds(tail, _N - tail)])


@functools.partial(
    pl.kernel,
    out_type=tuple(jax.ShapeDtypeStruct((_N, _D), jnp.float32) for _ in range(2)),
    mesh=_mesh,
    scratch_types=[
        pltpu.VMEM((_C,), jnp.int32),
        pltpu.VMEM((_C,), jnp.int32),
        pltpu.VMEM((_C, _D), jnp.float32),
        pltpu.VMEM_SHARED((_N, _D), jnp.float32),
        pltpu.SemaphoreType.DMA,
    ],
)
def _agg_kernel(tbl0, s0, t0, tbl1, s1, t1, out0, out1, idx_s, idx_t, rows, acc, sem):
    cid = lax.axis_index("c")
    sid = lax.axis_index("s")

    def zrow(i, carry):
        for j in range(_D // 16):
            rows[i, pl.ds(j * 16, 16)] = jnp.zeros((16,), jnp.float32)
        return carry

    lax.fori_loop(0, _C, zrow, 0)
    for j in range(6):
        pltpu.sync_copy(rows.at[pl.ds(0, 104)],
                        acc.at[pl.ds(sid * _ZC + j * 104, 104)])

    @pl.when(sid == _NSUB - 1)
    def _():
        pltpu.sync_copy(rows.at[pl.ds(0, _N - _NSUB * _ZC)],
                        acc.at[pl.ds(_NSUB * _ZC, _N - _NSUB * _ZC)])

    plsc.subcore_barrier()

    for c, (tbl, s_h, t_h) in enumerate(((tbl0, s0, t0), (tbl1, s1, t1))):
        @pl.when(cid == c)
        def _():
            def body(k, carry):
                g = k * _NSUB + sid

                @pl.when(g < _NCH)
                def _():
                    pltpu.sync_copy(s_h.at[pl.ds(g * _C, _C)], idx_s)
                    pltpu.sync_copy(t_h.at[pl.ds(g * _C, _C)], idx_t)
                    pltpu.async_copy(tbl.at[idx_s], rows, sem).wait()
                    pltpu.sync_copy(rows, acc.at[idx_t], add=True)
                return carry

            lax.fori_loop(0, _KMAX, body, 0)

    plsc.subcore_barrier()

    for c, out in enumerate((out0, out1)):
        @pl.when(cid == c)
        def _():
            for j in range(6):
                sl = pl.ds(sid * _ZC + j * 104, 104)
                pltpu.sync_copy(acc.at[sl], rows.at[pl.ds(0, 104)])
                pltpu.sync_copy(rows.at[pl.ds(0, 104)], out.at[sl])

            @pl.when(sid == _NSUB - 1)
            def _():
                sl = pl.ds(_NSUB * _ZC, _N - _NSUB * _ZC)
                pltpu.sync_copy(acc.at[sl], rows.at[pl.ds(0, _N - _NSUB * _ZC)])
                pltpu.sync_copy(rows.at[pl.ds(0, _N - _NSUB * _ZC)], out.at[sl])


_BLK = 1000
_GRID = _N // _BLK


def _safe_rs(d):
    return jnp.where(d > 0.0, lax.rsqrt(jnp.maximum(d, 1.0)), 0.0)


def _mm0_body(x_ref, w_ref, dg_ref, o_ref):
    rs = _safe_rs(dg_ref[...])
    o_ref[...] = jnp.dot(x_ref[...], w_ref[...],
                         preferred_element_type=jnp.float32) * rs


_mm0 = pl.pallas_call(
    _mm0_body,
    grid=(_GRID,),
    in_specs=[
        pl.BlockSpec((_BLK, _D), lambda i: (i, 0)),
        pl.BlockSpec((_D, _D), lambda i: (0, 0)),
        pl.BlockSpec((_BLK, 1), lambda i: (i, 0)),
    ],
    out_specs=pl.BlockSpec((_BLK, _D), lambda i: (i, 0)),
    out_shape=jax.ShapeDtypeStruct((_N, _D), jnp.float32),
)


def _mm1_body(a_ref, w_ref, din_ref, dout_ref, o_ref):
    x = jnp.maximum(a_ref[...] * _safe_rs(din_ref[...]), 0.0)
    o_ref[...] = jnp.dot(x, w_ref[...],
                         preferred_element_type=jnp.float32) * _safe_rs(dout_ref[...])


_mm1 = pl.pallas_call(
    _mm1_body,
    grid=(_GRID,),
    in_specs=[
        pl.BlockSpec((_BLK, _D), lambda i: (i, 0)),
        pl.BlockSpec((_D, _D), lambda i: (0, 0)),
        pl.BlockSpec((_BLK, 1), lambda i: (i, 0)),
        pl.BlockSpec((_BLK, 1), lambda i: (i, 0)),
    ],
    out_specs=pl.BlockSpec((_BLK, _D), lambda i: (i, 0)),
    out_shape=jax.ShapeDtypeStruct((_N, _D), jnp.float32),
)


def _fin_body(a_ref, din_ref, wl_ref, bl_ref, x_ref, o_ref):
    x = jnp.maximum(a_ref[...] * _safe_rs(din_ref[...]), 0.0)
    x_ref[...] = x
    o_ref[...] = jnp.sum(x * wl_ref[...], axis=1, keepdims=True) + bl_ref[0, 0]


_fin = pl.pallas_call(
    _fin_body,
    grid=(_GRID,),
    in_specs=[
        pl.BlockSpec((_BLK, _D), lambda i: (i, 0)),
        pl.BlockSpec((_BLK, 1), lambda i: (i, 0)),
        pl.BlockSpec((1, _D), lambda i: (0, 0)),
        pl.BlockSpec((1, 1), lambda i: (0, 0)),
    ],
    out_specs=[
        pl.BlockSpec((_BLK, _D), lambda i: (i, 0)),
        pl.BlockSpec((_BLK, 1), lambda i: (i, 0)),
    ],
    out_shape=[
        jax.ShapeDtypeStruct((_N, _D), jnp.float32),
        jax.ShapeDtypeStruct((_N, 1), jnp.float32),
    ],
)


def kernel(x_user, x_item, edge_index_ui, edge_index_iu,
           W0_ui_s, W0_ui_t, W0_iu_s, W0_iu_t,
           W1_ui_s, W1_ui_t, W1_iu_s, W1_iu_t,
           Wl_u, bl_u, Wl_i, bl_i):
    s_ui = edge_index_ui[0].astype(jnp.int32)
    t_ui = edge_index_ui[1].astype(jnp.int32)
    s_iu = edge_index_iu[0].astype(jnp.int32)
    t_iu = edge_index_iu[1].astype(jnp.int32)

    ds_ui, dt_ui, ds_iu, dt_iu = _deg_kernel(s_ui, t_ui, s_iu, t_iu)
    ds_ui = ds_ui.reshape(_N, 1)
    dt_ui = dt_ui.reshape(_N, 1)
    ds_iu = ds_iu.reshape(_N, 1)
    dt_iu = dt_iu.reshape(_N, 1)

    su0 = _mm0(x_user, W0_ui_s, ds_ui)
    si0 = _mm0(x_item, W0_iu_s, ds_iu)
    agg_i0, agg_u0 = _agg_kernel(su0, s_ui, t_ui, si0, s_iu, t_iu)

    su1 = _mm1(agg_u0, W1_ui_s, dt_iu, ds_ui)
    si1 = _mm1(agg_i0, W1_iu_s, dt_ui, ds_iu)
    agg_i1, agg_u1 = _agg_kernel(su1, s_ui, t_ui, si1, s_iu, t_iu)

    xu, out_u = _fin(agg_u1, dt_iu, Wl_u.reshape(1, _D), bl_u.reshape(1, 1))
    xi, out_i = _fin(agg_i1, dt_ui, Wl_i.reshape(1, _D), bl_i.reshape(1, 1))
    return (xu, xi, out_u, out_i)


# speedup vs baseline: 21.2851x; 2.0844x over previous
import functools

import jax
import jax.numpy as jnp
from jax import lax
from jax.experimental import pallas as pl
from jax.experimental.pallas import tpu as pltpu
from jax.experimental.pallas import tpu_sc as plsc

_N = 10000
_D = 128
_E = 320000
_C = 128
_NCH = _E // _C
_NSUB = 16
_CPT = 160
_CPAD = _NSUB * _CPT
_NBUF = 4
_ZC = 624
_TAIL = _N - _NSUB * _ZC

_mesh = plsc.VectorSubcoreMesh(core_axis_name="c", subcore_axis_name="s")


def _nch_for(sid):
    return jnp.where(sid == _NSUB - 1, _NCH - (_NSUB - 1) * _CPT, _CPT)


@functools.partial(
    pl.kernel,
    out_type=tuple(jax.ShapeDtypeStruct((_N,), jnp.float32) for _ in range(4)),
    mesh=_mesh,
    scratch_types=[
        pltpu.VMEM((_CPT, _C), jnp.int32),
        pltpu.VMEM((_CPT, _C), jnp.int32),
        pltpu.VMEM((_C,), jnp.float32),
        pltpu.VMEM((_ZC,), jnp.float32),
        pltpu.VMEM_SHARED((_N,), jnp.float32),
        pltpu.VMEM_SHARED((_N,), jnp.float32),
        pltpu.SemaphoreType.DMA,
    ],
)
def _deg_kernel(s0, t0, s1, t1, ds0, dt0, ds1, dt1,
                sidx, tidx, ones, zbuf, sh_s, sh_t, sem):
    cid = lax.axis_index("c")
    sid = lax.axis_index("s")
    nch = _nch_for(sid)
    base = sid * _ZC

    for i in range(_C // 16):
        ones[pl.ds(i * 16, 16)] = jnp.full((16,), 1.0, jnp.float32)
    for i in range(_ZC // 16):
        zbuf[pl.ds(i * 16, 16)] = jnp.zeros((16,), jnp.float32)

    for c, (s_h, t_h) in enumerate(((s0, t0), (s1, t1))):
        @pl.when(cid == c)
        def _():
            pltpu.sync_copy(s_h.at[pl.ds(sid * _CPT, _CPT)], sidx)
            pltpu.sync_copy(t_h.at[pl.ds(sid * _CPT, _CPT)], tidx)

    pltpu.sync_copy(zbuf, sh_s.at[pl.ds(base, _ZC)])
    pltpu.sync_copy(zbuf, sh_t.at[pl.ds(base, _ZC)])

    @pl.when(sid == _NSUB - 1)
    def _():
        pltpu.sync_copy(zbuf.at[pl.ds(0, _TAIL)], sh_s.at[pl.ds(_NSUB * _ZC, _TAIL)])
        pltpu.sync_copy(zbuf.at[pl.ds(0, _TAIL)], sh_t.at[pl.ds(_NSUB * _ZC, _TAIL)])

    plsc.subcore_barrier()

    def _drain_one(out_hbm):
        pltpu.make_async_copy(out_hbm.at[pl.ds(0, _C)], ones, sem).wait()

    for c, (o_s, o_t) in enumerate(((ds0, dt0), (ds1, dt1))):
        @pl.when(cid == c)
        def _():
            def fire(j, carry):
                pltpu.async_copy(ones, sh_s.at[sidx.at[j]], sem, add=True)
                pltpu.async_copy(ones, sh_t.at[tidx.at[j]], sem, add=True)

                @pl.when(j >= _NBUF)
                def _():
                    _drain_one(o_s)
                    _drain_one(o_s)
                return carry

            lax.fori_loop(0, nch, fire, 0)

            def drain(j, carry):
                _drain_one(o_s)
                _drain_one(o_s)
                return carry

            lax.fori_loop(0, jnp.minimum(nch, _NBUF), drain, 0)

    plsc.subcore_barrier()

    for c, (o_s, o_t) in enumerate(((ds0, dt0), (ds1, dt1))):
        @pl.when(cid == c)
        def _():
            for sh, out in ((sh_s, o_s), (sh_t, o_t)):
                pltpu.sync_copy(sh.at[pl.ds(base, _ZC)], zbuf)
                pltpu.sync_copy(zbuf, out.at[pl.ds(base, _ZC)])

                @pl.when(sid == _NSUB - 1)
                def _():
                    pltpu.sync_copy(sh.at[pl.ds(_NSUB * _ZC, _TAIL)], zbuf.at[pl.ds(0, _TAIL)])
                    pltpu.sync_copy(zbuf.at[pl.ds(0, _TAIL)], out.at[pl.ds(_NSUB * _ZC, _TAIL)])


_KM = (_NCH + _NSUB - 1) // _NSUB


@functools.partial(
    pl.kernel,
    out_type=tuple(jax.ShapeDtypeStruct((_N, _D), jnp.float32) for _ in range(2)),
    mesh=_mesh,
    scratch_types=[
        [pltpu.VMEM((_C,), jnp.int32) for _ in range(2)],
        [pltpu.VMEM((_C,), jnp.int32) for _ in range(2)],
        [pltpu.VMEM((_C, _D), jnp.float32) for _ in range(2)],
        [pltpu.SemaphoreType.DMA for _ in range(2)],
        [pltpu.SemaphoreType.DMA for _ in range(2)],
        pltpu.VMEM_SHARED((_N, _D), jnp.float32),
    ],
)
def _agg_kernel(tbl0, s0, t0, tbl1, s1, t1, out0, out1,
                isx, itx, rows, semi, semg, acc):
    cid = lax.axis_index("c")
    sid = lax.axis_index("s")

    def valid(k):
        return jnp.logical_and(k <= _KM - 1, k * _NSUB + sid < _NCH)

    def zrow(i, carry):
        for j in range(_D // 16):
            rows[0][i, pl.ds(j * 16, 16)] = jnp.zeros((16,), jnp.float32)
        return carry

    lax.fori_loop(0, _C, zrow, 0)

    for j in range(6):
        pltpu.sync_copy(rows[0].at[pl.ds(0, 104)],
                        acc.at[pl.ds(sid * _ZC + j * 104, 104)])

    @pl.when(sid == _NSUB - 1)
    def _():
        pltpu.sync_copy(rows[0].at[pl.ds(0, _TAIL)],
                        acc.at[pl.ds(_NSUB * _ZC, _TAIL)])

    plsc.subcore_barrier()

    for c, (tbl, s_h, t_h) in enumerate(((tbl0, s0, t0), (tbl1, s1, t1))):
        @pl.when(cid == c)
        def _():
            def issue_idx(k, b):
                g = k * _NSUB + sid
                pltpu.async_copy(s_h.at[pl.ds(g * _C, _C)], isx[b], semi[b])
                pltpu.async_copy(t_h.at[pl.ds(g * _C, _C)], itx[b], semi[b])

            def wait_idx(b):
                pltpu.make_async_copy(s_h.at[pl.ds(0, _C)], isx[b], semi[b]).wait()
                pltpu.make_async_copy(t_h.at[pl.ds(0, _C)], itx[b], semi[b]).wait()

            def step(k, b):
                @pl.when(valid(k))
                def _():
                    wait_idx(b)
                    pltpu.async_copy(tbl.at[isx[b]], rows[b], semg[b])

                @pl.when(valid(k - 1))
                def _():
                    pltpu.make_async_copy(tbl.at[pl.ds(0, _C)],
                                          rows[1 - b], semg[1 - b]).wait()
                    pltpu.sync_copy(rows[1 - b], acc.at[itx[1 - b]], add=True)

                    @pl.when(valid(k + 1))
                    def _():
                        issue_idx(k + 1, 1 - b)

            issue_idx(0, 0)

            @pl.when(valid(1))
            def _():
                issue_idx(1, 1)
            wait_idx(0)
            pltpu.async_copy(tbl.at[isx[0]], rows[0], semg[0])

            def pair(p, carry):
                step(2 * p + 1, 1)
                step(2 * p + 2, 0)
                return carry

            lax.fori_loop(0, (_KM + 2) // 2, pair, 0)

    plsc.subcore_barrier()

    for c, out in enumerate((out0, out1)):
        @pl.when(cid == c)
        def _():
            for j in range(6):
                sl = pl.ds(sid * _ZC + j * 104, 104)
                pltpu.sync_copy(acc.at[sl], rows[j % 2].at[pl.ds(0, 104)])
                pltpu.sync_copy(rows[j % 2].at[pl.ds(0, 104)], out.at[sl])

            @pl.when(sid == _NSUB - 1)
            def _():
                sl = pl.ds(_NSUB * _ZC, _TAIL)
                pltpu.sync_copy(acc.at[sl], rows[0].at[pl.ds(0, _TAIL)])
                pltpu.sync_copy(rows[0].at[pl.ds(0, _TAIL)], out.at[sl])


_BLK = 1000
_GRID = _N // _BLK


def _safe_rs(d):
    return jnp.where(d > 0.0, lax.rsqrt(jnp.maximum(d, 1.0)), 0.0)


def _mm0_body(x_ref, w_ref, dg_ref, o_ref):
    rs = _safe_rs(dg_ref[...])
    o_ref[...] = jnp.dot(x_ref[...], w_ref[...],
                         preferred_element_type=jnp.float32) * rs


_mm0 = pl.pallas_call(
    _mm0_body,
    grid=(_GRID,),
    in_specs=[
        pl.BlockSpec((_BLK, _D), lambda i: (i, 0)),
        pl.BlockSpec((_D, _D), lambda i: (0, 0)),
        pl.BlockSpec((_BLK, 1), lambda i: (i, 0)),
    ],
    out_specs=pl.BlockSpec((_BLK, _D), lambda i: (i, 0)),
    out_shape=jax.ShapeDtypeStruct((_N, _D), jnp.float32),
)


def _mm1_body(a_ref, w_ref, din_ref, dout_ref, o_ref):
    x = jnp.maximum(a_ref[...] * _safe_rs(din_ref[...]), 0.0)
    o_ref[...] = jnp.dot(x, w_ref[...],
                         preferred_element_type=jnp.float32) * _safe_rs(dout_ref[...])


_mm1 = pl.pallas_call(
    _mm1_body,
    grid=(_GRID,),
    in_specs=[
        pl.BlockSpec((_BLK, _D), lambda i: (i, 0)),
        pl.BlockSpec((_D, _D), lambda i: (0, 0)),
        pl.BlockSpec((_BLK, 1), lambda i: (i, 0)),
        pl.BlockSpec((_BLK, 1), lambda i: (i, 0)),
    ],
    out_specs=pl.BlockSpec((_BLK, _D), lambda i: (i, 0)),
    out_shape=jax.ShapeDtypeStruct((_N, _D), jnp.float32),
)


def _fin_body(a_ref, din_ref, wl_ref, bl_ref, x_ref, o_ref):
    x = jnp.maximum(a_ref[...] * _safe_rs(din_ref[...]), 0.0)
    x_ref[...] = x
    o_ref[...] = jnp.sum(x * wl_ref[...], axis=1, keepdims=True) + bl_ref[0, 0]


_fin = pl.pallas_call(
    _fin_body,
    grid=(_GRID,),
    in_specs=[
        pl.BlockSpec((_BLK, _D), lambda i: (i, 0)),
        pl.BlockSpec((_BLK, 1), lambda i: (i, 0)),
        pl.BlockSpec((1, _D), lambda i: (0, 0)),
        pl.BlockSpec((1, 1), lambda i: (0, 0)),
    ],
    out_specs=[
        pl.BlockSpec((_BLK, _D), lambda i: (i, 0)),
        pl.BlockSpec((_BLK, 1), lambda i: (i, 0)),
    ],
    out_shape=[
        jax.ShapeDtypeStruct((_N, _D), jnp.float32),
        jax.ShapeDtypeStruct((_N, 1), jnp.float32),
    ],
)


def kernel(x_user, x_item, edge_index_ui, edge_index_iu,
           W0_ui_s, W0_ui_t, W0_iu_s, W0_iu_t,
           W1_ui_s, W1_ui_t, W1_iu_s, W1_iu_t,
           Wl_u, bl_u, Wl_i, bl_i):
    def chunked(v):
        v = v.reshape(_NCH, _C)
        return jnp.pad(v, ((0, _CPAD - _NCH), (0, 0)))

    s_ui = edge_index_ui[0].astype(jnp.int32)
    t_ui = edge_index_ui[1].astype(jnp.int32)
    s_iu = edge_index_iu[0].astype(jnp.int32)
    t_iu = edge_index_iu[1].astype(jnp.int32)

    ds_ui, dt_ui, ds_iu, dt_iu = _deg_kernel(
        chunked(s_ui), chunked(t_ui), chunked(s_iu), chunked(t_iu))
    ds_ui = ds_ui.reshape(_N, 1)
    dt_ui = dt_ui.reshape(_N, 1)
    ds_iu = ds_iu.reshape(_N, 1)
    dt_iu = dt_iu.reshape(_N, 1)

    su0 = _mm0(x_user, W0_ui_s, ds_ui)
    si0 = _mm0(x_item, W0_iu_s, ds_iu)
    agg_i0, agg_u0 = _agg_kernel(su0, s_ui, t_ui, si0, s_iu, t_iu)

    su1 = _mm1(agg_u0, W1_ui_s, dt_iu, ds_ui)
    si1 = _mm1(agg_i0, W1_iu_s, dt_ui, ds_iu)
    agg_i1, agg_u1 = _agg_kernel(su1, s_ui, t_ui, si1, s_iu, t_iu)

    xu, out_u = _fin(agg_u1, dt_iu, Wl_u.reshape(1, _D), bl_u.reshape(1, 1))
    xi, out_i = _fin(agg_i1, dt_ui, Wl_i.reshape(1, _D), bl_i.reshape(1, 1))
    return (xu, xi, out_u, out_i)


# speedup vs baseline: 23.8652x; 1.1212x over previous
import functools

import jax
import jax.numpy as jnp
from jax import lax
from jax.experimental import pallas as pl
from jax.experimental.pallas import tpu as pltpu
from jax.experimental.pallas import tpu_sc as plsc

_N = 10000
_D = 128
_E = 320000
_C = 128
_NCH = _E // _C
_NSUB = 16
_CPT = 160
_CPAD = _NSUB * _CPT
_NBUF = 4
_ZC = 624
_TAIL = _N - _NSUB * _ZC

_mesh = plsc.VectorSubcoreMesh(core_axis_name="c", subcore_axis_name="s")


def _nch_for(sid):
    return jnp.where(sid == _NSUB - 1, _NCH - (_NSUB - 1) * _CPT, _CPT)


@functools.partial(
    pl.kernel,
    out_type=tuple(jax.ShapeDtypeStruct((_N,), jnp.float32) for _ in range(4)),
    mesh=_mesh,
    scratch_types=[
        pltpu.VMEM((_CPT, _C), jnp.int32),
        pltpu.VMEM((_CPT, _C), jnp.int32),
        pltpu.VMEM((_C,), jnp.float32),
        pltpu.VMEM((_ZC,), jnp.float32),
        pltpu.VMEM_SHARED((_N,), jnp.float32),
        pltpu.VMEM_SHARED((_N,), jnp.float32),
        pltpu.SemaphoreType.DMA,
    ],
)
def _deg_kernel(s0, t0, s1, t1, ds0, dt0, ds1, dt1,
                sidx, tidx, ones, zbuf, sh_s, sh_t, sem):
    cid = lax.axis_index("c")
    sid = lax.axis_index("s")
    nch = _nch_for(sid)
    base = sid * _ZC

    for i in range(_C // 16):
        ones[pl.ds(i * 16, 16)] = jnp.full((16,), 1.0, jnp.float32)
    for i in range(_ZC // 16):
        zbuf[pl.ds(i * 16, 16)] = jnp.zeros((16,), jnp.float32)

    for c, (s_h, t_h) in enumerate(((s0, t0), (s1, t1))):
        @pl.when(cid == c)
        def _():
            pltpu.sync_copy(s_h.at[pl.ds(sid * _CPT, _CPT)], sidx)
            pltpu.sync_copy(t_h.at[pl.ds(sid * _CPT, _CPT)], tidx)

    pltpu.sync_copy(zbuf, sh_s.at[pl.ds(base, _ZC)])
    pltpu.sync_copy(zbuf, sh_t.at[pl.ds(base, _ZC)])

    @pl.when(sid == _NSUB - 1)
    def _():
        pltpu.sync_copy(zbuf.at[pl.ds(0, _TAIL)], sh_s.at[pl.ds(_NSUB * _ZC, _TAIL)])
        pltpu.sync_copy(zbuf.at[pl.ds(0, _TAIL)], sh_t.at[pl.ds(_NSUB * _ZC, _TAIL)])

    plsc.subcore_barrier()

    def _drain_one(out_hbm):
        pltpu.make_async_copy(out_hbm.at[pl.ds(0, _C)], ones, sem).wait()

    for c, (o_s, o_t) in enumerate(((ds0, dt0), (ds1, dt1))):
        @pl.when(cid == c)
        def _():
            def fire(j, carry):
                pltpu.async_copy(ones, sh_s.at[sidx.at[j]], sem, add=True)
                pltpu.async_copy(ones, sh_t.at[tidx.at[j]], sem, add=True)

                @pl.when(j >= _NBUF)
                def _():
                    _drain_one(o_s)
                    _drain_one(o_s)
                return carry

            lax.fori_loop(0, nch, fire, 0)

            def drain(j, carry):
                _drain_one(o_s)
                _drain_one(o_s)
                return carry

            lax.fori_loop(0, jnp.minimum(nch, _NBUF), drain, 0)

    plsc.subcore_barrier()

    for c, (o_s, o_t) in enumerate(((ds0, dt0), (ds1, dt1))):
        @pl.when(cid == c)
        def _():
            for sh, out in ((sh_s, o_s), (sh_t, o_t)):
                pltpu.sync_copy(sh.at[pl.ds(base, _ZC)], zbuf)
                pltpu.sync_copy(zbuf, out.at[pl.ds(base, _ZC)])

                @pl.when(sid == _NSUB - 1)
                def _():
                    pltpu.sync_copy(sh.at[pl.ds(_NSUB * _ZC, _TAIL)], zbuf.at[pl.ds(0, _TAIL)])
                    pltpu.sync_copy(zbuf.at[pl.ds(0, _TAIL)], out.at[pl.ds(_NSUB * _ZC, _TAIL)])


_KM = (_NCH + _NSUB - 1) // _NSUB


@functools.partial(
    pl.kernel,
    out_type=tuple(jax.ShapeDtypeStruct((_N, _D), jnp.float32) for _ in range(2)),
    mesh=_mesh,
    scratch_types=[
        [pltpu.VMEM((_C,), jnp.int32) for _ in range(4)],
        [pltpu.VMEM((_C,), jnp.int32) for _ in range(4)],
        [pltpu.VMEM((_C, _D), jnp.float32) for _ in range(2)],
        [pltpu.SemaphoreType.DMA for _ in range(4)],
        [pltpu.SemaphoreType.DMA for _ in range(2)],
        [pltpu.SemaphoreType.DMA for _ in range(2)],
        pltpu.VMEM_SHARED((_N, _D), jnp.float32),
    ],
)
def _agg_kernel(tbl0, s0, t0, tbl1, s1, t1, out0, out1,
                isx, itx, rows, semi, semg, sems, acc):
    cid = lax.axis_index("c")
    sid = lax.axis_index("s")

    def valid(k):
        in_range = jnp.logical_and(k >= 0, k <= _KM - 1)
        return jnp.logical_and(in_range, k * _NSUB + sid < _NCH)

    def zrow(i, carry):
        for j in range(_D // 16):
            rows[0][i, pl.ds(j * 16, 16)] = jnp.zeros((16,), jnp.float32)
        return carry

    lax.fori_loop(0, _C, zrow, 0)

    for j in range(6):
        pltpu.sync_copy(rows[0].at[pl.ds(0, 104)],
                        acc.at[pl.ds(sid * _ZC + j * 104, 104)])

    @pl.when(sid == _NSUB - 1)
    def _():
        pltpu.sync_copy(rows[0].at[pl.ds(0, _TAIL)],
                        acc.at[pl.ds(_NSUB * _ZC, _TAIL)])

    plsc.subcore_barrier()

    for c, (tbl, s_h, t_h) in enumerate(((tbl0, s0, t0), (tbl1, s1, t1))):
        @pl.when(cid == c)
        def _():
            def issue_idx(k, ib):
                g = k * _NSUB + sid
                pltpu.async_copy(s_h.at[pl.ds(g * _C, _C)], isx[ib], semi[ib])
                pltpu.async_copy(t_h.at[pl.ds(g * _C, _C)], itx[ib], semi[ib])

            def wait_idx(ib):
                pltpu.make_async_copy(s_h.at[pl.ds(0, _C)], isx[ib], semi[ib]).wait()
                pltpu.make_async_copy(t_h.at[pl.ds(0, _C)], itx[ib], semi[ib]).wait()

            def drain_scatter(rb, ib):
                pltpu.make_async_copy(rows[rb], acc.at[itx[ib]], sems[rb]).wait()

            def step(k, o):
                rb, ib = o % 2, o % 4

                @pl.when(valid(k))
                def _():
                    wait_idx(ib)

                    @pl.when(valid(k - 2))
                    def _():
                        drain_scatter(rb, (o + 2) % 4)
                    pltpu.async_copy(tbl.at[isx[ib]], rows[rb], semg[rb])

                @pl.when(valid(k - 1))
                def _():
                    pltpu.make_async_copy(tbl.at[pl.ds(0, _C)],
                                          rows[1 - rb], semg[1 - rb]).wait()
                    pltpu.async_copy(rows[1 - rb], acc.at[itx[(o - 1) % 4]],
                                     sems[1 - rb], add=True)

                    @pl.when(valid(k + 2))
                    def _():
                        issue_idx(k + 2, (o + 2) % 4)

            issue_idx(0, 0)

            @pl.when(valid(1))
            def _():
                issue_idx(1, 1)

            @pl.when(valid(2))
            def _():
                issue_idx(2, 2)
            wait_idx(0)
            pltpu.async_copy(tbl.at[isx[0]], rows[0], semg[0])

            def quad(p, carry):
                k = 4 * p
                step(k + 1, 1)
                step(k + 2, 2)
                step(k + 3, 3)
                step(k + 4, 4)
                return carry

            lax.fori_loop(0, (_KM + 3) // 4, quad, 0)

            for kt in (_KM - 3, _KM - 2, _KM - 1):
                @pl.when(jnp.logical_and(valid(kt),
                                         jnp.logical_not(valid(kt + 2))))
                def _(kt=kt):
                    drain_scatter(kt % 2, kt % 4)

    plsc.subcore_barrier()

    for c, out in enumerate((out0, out1)):
        @pl.when(cid == c)
        def _():
            for j in range(6):
                sl = pl.ds(sid * _ZC + j * 104, 104)
                pltpu.sync_copy(acc.at[sl], rows[j % 2].at[pl.ds(0, 104)])
                pltpu.sync_copy(rows[j % 2].at[pl.ds(0, 104)], out.at[sl])

            @pl.when(sid == _NSUB - 1)
            def _():
                sl = pl.ds(_NSUB * _ZC, _TAIL)
                pltpu.sync_copy(acc.at[sl], rows[0].at[pl.ds(0, _TAIL)])
                pltpu.sync_copy(rows[0].at[pl.ds(0, _TAIL)], out.at[sl])


_BLK = 1000
_GRID = _N // _BLK


def _safe_rs(d):
    return jnp.where(d > 0.0, lax.rsqrt(jnp.maximum(d, 1.0)), 0.0)


def _mm0_body(x_ref, w_ref, dg_ref, o_ref):
    rs = _safe_rs(dg_ref[...])
    o_ref[...] = jnp.dot(x_ref[...], w_ref[...],
                         preferred_element_type=jnp.float32) * rs


_mm0 = pl.pallas_call(
    _mm0_body,
    grid=(_GRID,),
    in_specs=[
        pl.BlockSpec((_BLK, _D), lambda i: (i, 0)),
        pl.BlockSpec((_D, _D), lambda i: (0, 0)),
        pl.BlockSpec((_BLK, 1), lambda i: (i, 0)),
    ],
    out_specs=pl.BlockSpec((_BLK, _D), lambda i: (i, 0)),
    out_shape=jax.ShapeDtypeStruct((_N, _D), jnp.float32),
)


def _mm1_body(a_ref, w_ref, din_ref, dout_ref, o_ref):
    x = jnp.maximum(a_ref[...] * _safe_rs(din_ref[...]), 0.0)
    o_ref[...] = jnp.dot(x, w_ref[...],
                         preferred_element_type=jnp.float32) * _safe_rs(dout_ref[...])


_mm1 = pl.pallas_call(
    _mm1_body,
    grid=(_GRID,),
    in_specs=[
        pl.BlockSpec((_BLK, _D), lambda i: (i, 0)),
        pl.BlockSpec((_D, _D), lambda i: (0, 0)),
        pl.BlockSpec((_BLK, 1), lambda i: (i, 0)),
        pl.BlockSpec((_BLK, 1), lambda i: (i, 0)),
    ],
    out_specs=pl.BlockSpec((_BLK, _D), lambda i: (i, 0)),
    out_shape=jax.ShapeDtypeStruct((_N, _D), jnp.float32),
)


def _fin_body(a_ref, din_ref, wl_ref, bl_ref, x_ref, o_ref):
    x = jnp.maximum(a_ref[...] * _safe_rs(din_ref[...]), 0.0)
    x_ref[...] = x
    o_ref[...] = jnp.sum(x * wl_ref[...], axis=1, keepdims=True) + bl_ref[0, 0]


_fin = pl.pallas_call(
    _fin_body,
    grid=(_GRID,),
    in_specs=[
        pl.BlockSpec((_BLK, _D), lambda i: (i, 0)),
        pl.BlockSpec((_BLK, 1), lambda i: (i, 0)),
        pl.BlockSpec((1, _D), lambda i: (0, 0)),
        pl.BlockSpec((1, 1), lambda i: (0, 0)),
    ],
    out_specs=[
        pl.BlockSpec((_BLK, _D), lambda i: (i, 0)),
        pl.BlockSpec((_BLK, 1), lambda i: (i, 0)),
    ],
    out_shape=[
        jax.ShapeDtypeStruct((_N, _D), jnp.float32),
        jax.ShapeDtypeStruct((_N, 1), jnp.float32),
    ],
)


def kernel(x_user, x_item, edge_index_ui, edge_index_iu,
           W0_ui_s, W0_ui_t, W0_iu_s, W0_iu_t,
           W1_ui_s, W1_ui_t, W1_iu_s, W1_iu_t,
           Wl_u, bl_u, Wl_i, bl_i):
    def chunked(v):
        v = v.reshape(_NCH, _C)
        return jnp.pad(v, ((0, _CPAD - _NCH), (0, 0)))

    s_ui = edge_index_ui[0].astype(jnp.int32)
    t_ui = edge_index_ui[1].astype(jnp.int32)
    s_iu = edge_index_iu[0].astype(jnp.int32)
    t_iu = edge_index_iu[1].astype(jnp.int32)

    ds_ui, dt_ui, ds_iu, dt_iu = _deg_kernel(
        chunked(s_ui), chunked(t_ui), chunked(s_iu), chunked(t_iu))
    ds_ui = ds_ui.reshape(_N, 1)
    dt_ui = dt_ui.reshape(_N, 1)
    ds_iu = ds_iu.reshape(_N, 1)
    dt_iu = dt_iu.reshape(_N, 1)

    su0 = _mm0(x_user, W0_ui_s, ds_ui)
    si0 = _mm0(x_item, W0_iu_s, ds_iu)
    agg_i0, agg_u0 = _agg_kernel(su0, s_ui, t_ui, si0, s_iu, t_iu)

    su1 = _mm1(agg_u0, W1_ui_s, dt_iu, ds_ui)
    si1 = _mm1(agg_i0, W1_iu_s, dt_ui, ds_iu)
    agg_i1, agg_u1 = _agg_kernel(su1, s_ui, t_ui, si1, s_iu, t_iu)

    xu, out_u = _fin(agg_u1, dt_iu, Wl_u.reshape(1, _D), bl_u.reshape(1, 1))
    xi, out_i = _fin(agg_i1, dt_ui, Wl_i.reshape(1, _D), bl_i.reshape(1, 1))
    return (xu, xi, out_u, out_i)


# speedup vs baseline: 23.8871x; 1.0009x over previous
import functools

import jax
import jax.numpy as jnp
from jax import lax
from jax.experimental import pallas as pl
from jax.experimental.pallas import tpu as pltpu
from jax.experimental.pallas import tpu_sc as plsc

_N = 10000
_D = 128
_E = 320000
_C = 128
_NCH = _E // _C
_NSUB = 16
_CPT = 160
_CPAD = _NSUB * _CPT
_NBUF = 4
_ZC = 624
_TAIL = _N - _NSUB * _ZC

_mesh = plsc.VectorSubcoreMesh(core_axis_name="c", subcore_axis_name="s")


def _nch_for(sid):
    return jnp.where(sid == _NSUB - 1, _NCH - (_NSUB - 1) * _CPT, _CPT)


@functools.partial(
    pl.kernel,
    out_type=tuple(jax.ShapeDtypeStruct((_N,), jnp.float32) for _ in range(4)),
    mesh=_mesh,
    scratch_types=[
        pltpu.VMEM((_CPT, _C), jnp.int32),
        pltpu.VMEM((_CPT, _C), jnp.int32),
        pltpu.VMEM((_C,), jnp.float32),
        pltpu.VMEM((_ZC,), jnp.float32),
        pltpu.VMEM_SHARED((_N,), jnp.float32),
        pltpu.VMEM_SHARED((_N,), jnp.float32),
        pltpu.SemaphoreType.DMA,
    ],
)
def _deg_kernel(s0, t0, s1, t1, ds0, dt0, ds1, dt1,
                sidx, tidx, ones, zbuf, sh_s, sh_t, sem):
    cid = lax.axis_index("c")
    sid = lax.axis_index("s")
    nch = _nch_for(sid)
    base = sid * _ZC

    for i in range(_C // 16):
        ones[pl.ds(i * 16, 16)] = jnp.full((16,), 1.0, jnp.float32)
    for i in range(_ZC // 16):
        zbuf[pl.ds(i * 16, 16)] = jnp.zeros((16,), jnp.float32)

    for c, (s_h, t_h) in enumerate(((s0, t0), (s1, t1))):
        @pl.when(cid == c)
        def _():
            pltpu.sync_copy(s_h.at[pl.ds(sid * _CPT, _CPT)], sidx)
            pltpu.sync_copy(t_h.at[pl.ds(sid * _CPT, _CPT)], tidx)

    pltpu.sync_copy(zbuf, sh_s.at[pl.ds(base, _ZC)])
    pltpu.sync_copy(zbuf, sh_t.at[pl.ds(base, _ZC)])

    @pl.when(sid == _NSUB - 1)
    def _():
        pltpu.sync_copy(zbuf.at[pl.ds(0, _TAIL)], sh_s.at[pl.ds(_NSUB * _ZC, _TAIL)])
        pltpu.sync_copy(zbuf.at[pl.ds(0, _TAIL)], sh_t.at[pl.ds(_NSUB * _ZC, _TAIL)])

    plsc.subcore_barrier()

    def _drain_one(out_hbm):
        pltpu.make_async_copy(out_hbm.at[pl.ds(0, _C)], ones, sem).wait()

    for c, (o_s, o_t) in enumerate(((ds0, dt0), (ds1, dt1))):
        @pl.when(cid == c)
        def _():
            def fire(j, carry):
                pltpu.async_copy(ones, sh_s.at[sidx.at[j]], sem, add=True)
                pltpu.async_copy(ones, sh_t.at[tidx.at[j]], sem, add=True)

                @pl.when(j >= _NBUF)
                def _():
                    _drain_one(o_s)
                    _drain_one(o_s)
                return carry

            lax.fori_loop(0, nch, fire, 0)

            def drain(j, carry):
                _drain_one(o_s)
                _drain_one(o_s)
                return carry

            lax.fori_loop(0, jnp.minimum(nch, _NBUF), drain, 0)

    plsc.subcore_barrier()

    for c, (o_s, o_t) in enumerate(((ds0, dt0), (ds1, dt1))):
        @pl.when(cid == c)
        def _():
            for sh, out in ((sh_s, o_s), (sh_t, o_t)):
                pltpu.sync_copy(sh.at[pl.ds(base, _ZC)], zbuf)
                pltpu.sync_copy(zbuf, out.at[pl.ds(base, _ZC)])

                @pl.when(sid == _NSUB - 1)
                def _():
                    pltpu.sync_copy(sh.at[pl.ds(_NSUB * _ZC, _TAIL)], zbuf.at[pl.ds(0, _TAIL)])
                    pltpu.sync_copy(zbuf.at[pl.ds(0, _TAIL)], out.at[pl.ds(_NSUB * _ZC, _TAIL)])


_KM = (_NCH + _NSUB - 1) // _NSUB


@functools.partial(
    pl.kernel,
    out_type=tuple(jax.ShapeDtypeStruct((_N, _D), jnp.float32) for _ in range(2)),
    mesh=_mesh,
    scratch_types=[
        [pltpu.VMEM((_C,), jnp.int32) for _ in range(4)],
        [pltpu.VMEM((_C,), jnp.int32) for _ in range(4)],
        [pltpu.VMEM((_C, _D), jnp.float32) for _ in range(2)],
        [pltpu.SemaphoreType.DMA for _ in range(4)],
        [pltpu.SemaphoreType.DMA for _ in range(2)],
        [pltpu.SemaphoreType.DMA for _ in range(2)],
        pltpu.VMEM_SHARED((_N, _D), jnp.float32),
    ],
)
def _agg_kernel(tbl0, s0, t0, tbl1, s1, t1, out0, out1,
                isx, itx, rows, semi, semg, sems, acc):
    cid = lax.axis_index("c")
    sid = lax.axis_index("s")

    def valid(k):
        in_range = jnp.logical_and(k >= 0, k <= _KM - 1)
        return jnp.logical_and(in_range, k * _NSUB + sid < _NCH)

    def zrow(i, carry):
        for j in range(_D // 16):
            rows[0][i, pl.ds(j * 16, 16)] = jnp.zeros((16,), jnp.float32)
        return carry

    lax.fori_loop(0, _C, zrow, 0)

    for j in range(6):
        pltpu.sync_copy(rows[0].at[pl.ds(0, 104)],
                        acc.at[pl.ds(sid * _ZC + j * 104, 104)])

    @pl.when(sid == _NSUB - 1)
    def _():
        pltpu.sync_copy(rows[0].at[pl.ds(0, _TAIL)],
                        acc.at[pl.ds(_NSUB * _ZC, _TAIL)])

    plsc.subcore_barrier()

    for c, (tbl, s_h, t_h) in enumerate(((tbl0, s0, t0), (tbl1, s1, t1))):
        @pl.when(cid == c)
        def _():
            def issue_idx(k, ib):
                g = k * _NSUB + sid
                pltpu.async_copy(s_h.at[pl.ds(g * _C, _C)], isx[ib], semi[ib])
                pltpu.async_copy(t_h.at[pl.ds(g * _C, _C)], itx[ib], semi[ib])

            def wait_idx(ib):
                pltpu.make_async_copy(s_h.at[pl.ds(0, _C)], isx[ib], semi[ib]).wait()
                pltpu.make_async_copy(t_h.at[pl.ds(0, _C)], itx[ib], semi[ib]).wait()

            def drain_scatter(rb, ib):
                pltpu.make_async_copy(rows[rb], acc.at[itx[ib]], sems[rb]).wait()

            def step(k, o):
                rb, ib = o % 2, o % 4

                @pl.when(valid(k))
                def _():
                    wait_idx(ib)

                    @pl.when(valid(k - 2))
                    def _():
                        drain_scatter(rb, (o + 2) % 4)
                    pltpu.async_copy(tbl.at[isx[ib]], rows[rb], semg[rb])

                @pl.when(valid(k - 1))
                def _():
                    pltpu.make_async_copy(tbl.at[pl.ds(0, _C)],
                                          rows[1 - rb], semg[1 - rb]).wait()
                    pltpu.async_copy(rows[1 - rb], acc.at[itx[(o - 1) % 4]],
                                     sems[1 - rb], add=True)

                    @pl.when(valid(k + 2))
                    def _():
                        issue_idx(k + 2, (o + 2) % 4)

            issue_idx(0, 0)

            @pl.when(valid(1))
            def _():
                issue_idx(1, 1)

            @pl.when(valid(2))
            def _():
                issue_idx(2, 2)
            wait_idx(0)
            pltpu.async_copy(tbl.at[isx[0]], rows[0], semg[0])

            def quad(p, carry):
                k = 4 * p
                step(k + 1, 1)
                step(k + 2, 2)
                step(k + 3, 3)
                step(k + 4, 4)
                return carry

            lax.fori_loop(0, (_KM + 3) // 4, quad, 0)

            for kt in (_KM - 3, _KM - 2, _KM - 1):
                @pl.when(jnp.logical_and(valid(kt),
                                         jnp.logical_not(valid(kt + 2))))
                def _(kt=kt):
                    drain_scatter(kt % 2, kt % 4)

    plsc.subcore_barrier()

    for c, out in enumerate((out0, out1)):
        @pl.when(cid == c)
        def _():
            for j in range(6):
                sl = pl.ds(sid * _ZC + j * 104, 104)
                pltpu.sync_copy(acc.at[sl], rows[j % 2].at[pl.ds(0, 104)])
                pltpu.sync_copy(rows[j % 2].at[pl.ds(0, 104)], out.at[sl])

            @pl.when(sid == _NSUB - 1)
            def _():
                sl = pl.ds(_NSUB * _ZC, _TAIL)
                pltpu.sync_copy(acc.at[sl], rows[0].at[pl.ds(0, _TAIL)])
                pltpu.sync_copy(rows[0].at[pl.ds(0, _TAIL)], out.at[sl])


_BLK = 1000
_GRID = _N // _BLK


def _safe_rs(d):
    return jnp.where(d > 0.0, lax.rsqrt(jnp.maximum(d, 1.0)), 0.0)


def _mm0_body(x_ref, w_ref, dg_ref, o_ref):
    rs = _safe_rs(dg_ref[...])
    o_ref[...] = jnp.dot(x_ref[...], w_ref[...],
                         preferred_element_type=jnp.float32) * rs


_mm0 = pl.pallas_call(
    _mm0_body,
    grid=(_GRID,),
    in_specs=[
        pl.BlockSpec((_BLK, _D), lambda i: (i, 0)),
        pl.BlockSpec((_D, _D), lambda i: (0, 0)),
        pl.BlockSpec((_BLK, 1), lambda i: (i, 0)),
    ],
    out_specs=pl.BlockSpec((_BLK, _D), lambda i: (i, 0)),
    out_shape=jax.ShapeDtypeStruct((_N, _D), jnp.float32),
)


def _mm1_body(a_ref, w_ref, din_ref, dout_ref, o_ref):
    x = jnp.maximum(a_ref[...] * _safe_rs(din_ref[...]), 0.0)
    o_ref[...] = jnp.dot(x, w_ref[...],
                         preferred_element_type=jnp.float32) * _safe_rs(dout_ref[...])


_mm1 = pl.pallas_call(
    _mm1_body,
    grid=(_GRID,),
    in_specs=[
        pl.BlockSpec((_BLK, _D), lambda i: (i, 0)),
        pl.BlockSpec((_D, _D), lambda i: (0, 0)),
        pl.BlockSpec((_BLK, 1), lambda i: (i, 0)),
        pl.BlockSpec((_BLK, 1), lambda i: (i, 0)),
    ],
    out_specs=pl.BlockSpec((_BLK, _D), lambda i: (i, 0)),
    out_shape=jax.ShapeDtypeStruct((_N, _D), jnp.float32),
)


def _fin_body(a_ref, din_ref, wl_ref, bl_ref, x_ref, o_ref):
    x = jnp.maximum(a_ref[...] * _safe_rs(din_ref[...]), 0.0)
    x_ref[...] = x
    o_ref[...] = jnp.dot(x, wl_ref[...],
                         preferred_element_type=jnp.float32) + bl_ref[0, 0]


_fin = pl.pallas_call(
    _fin_body,
    grid=(_GRID,),
    in_specs=[
        pl.BlockSpec((_BLK, _D), lambda i: (i, 0)),
        pl.BlockSpec((_BLK, 1), lambda i: (i, 0)),
        pl.BlockSpec((_D, 1), lambda i: (0, 0)),
        pl.BlockSpec((1, 1), lambda i: (0, 0)),
    ],
    out_specs=[
        pl.BlockSpec((_BLK, _D), lambda i: (i, 0)),
        pl.BlockSpec((_BLK, 1), lambda i: (i, 0)),
    ],
    out_shape=[
        jax.ShapeDtypeStruct((_N, _D), jnp.float32),
        jax.ShapeDtypeStruct((_N, 1), jnp.float32),
    ],
)


def kernel(x_user, x_item, edge_index_ui, edge_index_iu,
           W0_ui_s, W0_ui_t, W0_iu_s, W0_iu_t,
           W1_ui_s, W1_ui_t, W1_iu_s, W1_iu_t,
           Wl_u, bl_u, Wl_i, bl_i):
    def chunked(v):
        v = v.reshape(_NCH, _C)
        return jnp.pad(v, ((0, _CPAD - _NCH), (0, 0)))

    s_ui = edge_index_ui[0].astype(jnp.int32)
    t_ui = edge_index_ui[1].astype(jnp.int32)
    s_iu = edge_index_iu[0].astype(jnp.int32)
    t_iu = edge_index_iu[1].astype(jnp.int32)

    ds_ui, dt_ui, ds_iu, dt_iu = _deg_kernel(
        chunked(s_ui), chunked(t_ui), chunked(s_iu), chunked(t_iu))
    ds_ui = ds_ui.reshape(_N, 1)
    dt_ui = dt_ui.reshape(_N, 1)
    ds_iu = ds_iu.reshape(_N, 1)
    dt_iu = dt_iu.reshape(_N, 1)

    su0 = _mm0(x_user, W0_ui_s, ds_ui)
    si0 = _mm0(x_item, W0_iu_s, ds_iu)
    agg_i0, agg_u0 = _agg_kernel(su0, s_ui, t_ui, si0, s_iu, t_iu)

    su1 = _mm1(agg_u0, W1_ui_s, dt_iu, ds_ui)
    si1 = _mm1(agg_i0, W1_iu_s, dt_ui, ds_iu)
    agg_i1, agg_u1 = _agg_kernel(su1, s_ui, t_ui, si1, s_iu, t_iu)

    xu, out_u = _fin(agg_u1, dt_iu, Wl_u, bl_u.reshape(1, 1))
    xi, out_i = _fin(agg_i1, dt_ui, Wl_i, bl_i.reshape(1, 1))
    return (xu, xi, out_u, out_i)
